# baseline (speedup 1.0000x reference)
import functools

import jax
import jax.numpy as jnp
from jax.experimental import pallas as pl
from jax.experimental.pallas import tpu as pltpu


def _routing_body(x_ref, wr_ref, br_ref, eid_ref):
    logits = jax.lax.dot_general(
        x_ref[...], wr_ref[...],
        dimension_numbers=(((1,), (1,)), ((), ())),
        preferred_element_type=jnp.float32,
    )
    logits = logits + br_ref[...]
    eid = jnp.argmax(logits, axis=1).astype(jnp.int32)
    eid_ref[...] = eid[:, None]


def _route(xf, Wr, br2):
    S = xf.shape[0]
    return pl.pallas_call(
        _routing_body,
        out_shape=jax.ShapeDtypeStruct((S, 1), jnp.int32),
    )(xf, Wr, br2)


def _moe_body(b_ref, e_ref, lo_ref, hi_ref,
              xs_ref, wg_ref, wu_ref, wd_ref, out_ref, *, bt):
    h = pl.program_id(0)
    w = pl.program_id(1)

    @pl.when((h == 0) & (w == 0))
    def _init():
        out_ref[...] = jnp.zeros_like(out_ref)

    b = b_ref[w]
    lo = lo_ref[w]
    hi = hi_ref[w]

    @pl.when(hi > lo)
    def _compute():
        xb = xs_ref[pl.ds(b * bt, bt), :]
        wg = wg_ref[0]
        wu = wu_ref[0]
        wd = wd_ref[0]
        dn = (((1,), (1,)), ((), ()))
        gate = jax.lax.dot_general(xb, wg, dn,
                                   preferred_element_type=jnp.float32,
                                   precision=jax.lax.Precision.HIGHEST)
        up = jax.lax.dot_general(xb, wu, dn,
                                 preferred_element_type=jnp.float32,
                                 precision=jax.lax.Precision.HIGHEST)
        act = gate * jax.nn.sigmoid(gate) * up
        contrib = jax.lax.dot_general(act, wd, dn,
                                      preferred_element_type=jnp.float32,
                                      precision=jax.lax.Precision.HIGHEST)
        iota = jax.lax.broadcasted_iota(jnp.int32, (bt, 1), 0)
        mask = ((iota >= lo) & (iota < hi)).astype(contrib.dtype)
        out_ref[pl.ds(b * bt, bt), :] += contrib * mask


def _grouped_swiglu(xs, Wgate, Wup, Wdown, b_ids, e_ids, lo, hi, *, bt, bh):
    S, D = xs.shape
    E, H, _ = Wgate.shape
    nh = H // bh
    nw = b_ids.shape[0]

    grid_spec = pltpu.PrefetchScalarGridSpec(
        num_scalar_prefetch=4,
        grid=(nh, nw),
        in_specs=[
            pl.BlockSpec((S, D), lambda h, w, b, e, l, u: (0, 0)),
            pl.BlockSpec((1, bh, D), lambda h, w, b, e, l, u: (e[w], h, 0)),
            pl.BlockSpec((1, bh, D), lambda h, w, b, e, l, u: (e[w], h, 0)),
            pl.BlockSpec((1, D, bh), lambda h, w, b, e, l, u: (e[w], 0, h)),
        ],
        out_specs=pl.BlockSpec((S, D), lambda h, w, b, e, l, u: (0, 0)),
    )
    return pl.pallas_call(
        functools.partial(_moe_body, bt=bt),
        grid_spec=grid_spec,
        out_shape=jax.ShapeDtypeStruct((S, D), jnp.float32),
        compiler_params=pltpu.CompilerParams(
            dimension_semantics=("arbitrary", "arbitrary"),
        ),
    )(b_ids, e_ids, lo, hi, xs, Wgate, Wup, Wdown)


def kernel(x, Wr, br, Wgate, Wup, Wdown):
    Bsz, S, D = x.shape
    E, H, _ = Wgate.shape
    BT = 128
    BH = 512
    NB = S // BT
    NW = NB + E - 1

    xf = x.reshape(S, D)
    eid = _route(xf, Wr, br.reshape(1, E))[:, 0]

    counts = jnp.sum(eid[:, None] == jnp.arange(E, dtype=jnp.int32)[None, :],
                     axis=0).astype(jnp.int32)
    offs_end = jnp.cumsum(counts).astype(jnp.int32)
    order = jnp.argsort(eid, stable=True)

    blk = jnp.arange(NB, dtype=jnp.int32)
    e_first = jnp.searchsorted(offs_end, blk * BT, side='right').astype(jnp.int32)
    e_last = jnp.searchsorted(offs_end, blk * BT + (BT - 1),
                              side='right').astype(jnp.int32)
    k = e_last - e_first + 1
    item_end = jnp.cumsum(k).astype(jnp.int32)
    total = item_end[-1]

    w = jnp.arange(NW, dtype=jnp.int32)
    b_ids = jnp.minimum(jnp.searchsorted(item_end, w, side='right'),
                        NB - 1).astype(jnp.int32)
    item_start = item_end[b_ids] - k[b_ids]
    e_ids = jnp.minimum(e_first[b_ids] + (w - item_start),
                        E - 1).astype(jnp.int32)
    valid = w < total

    row_s = offs_end[e_ids] - counts[e_ids]
    row_e = offs_end[e_ids]
    lo = jnp.clip(row_s - b_ids * BT, 0, BT).astype(jnp.int32)
    hi = jnp.clip(row_e - b_ids * BT, 0, BT).astype(jnp.int32)
    hi = jnp.where(valid, hi, lo)

    xs = jnp.take(xf, order, axis=0, unique_indices=True)
    outs = _grouped_swiglu(xs, Wgate, Wup, Wdown, b_ids, e_ids, lo, hi,
                           bt=BT, bh=BH)
    out = jnp.zeros_like(outs).at[order].set(outs, unique_indices=True)
    return out.reshape(Bsz, S, D)


# speedup vs baseline: 3.5450x; 3.5450x over previous
import functools

import jax
import jax.numpy as jnp
from jax.experimental import pallas as pl
from jax.experimental.pallas import tpu as pltpu


def _routing_body(x_ref, wr_ref, br_ref, eid_ref):
    logits = jax.lax.dot_general(
        x_ref[...], wr_ref[...],
        dimension_numbers=(((1,), (1,)), ((), ())),
        preferred_element_type=jnp.float32,
    )
    logits = logits + br_ref[...]
    eid = jnp.argmax(logits, axis=1).astype(jnp.int32)
    eid_ref[...] = eid[:, None]


def _route(xf, Wr, br2):
    S = xf.shape[0]
    return pl.pallas_call(
        _routing_body,
        out_shape=jax.ShapeDtypeStruct((S, 1), jnp.int32),
    )(xf, Wr, br2)


def _moe_body(b_ref, e_ref, lo_ref, hi_ref,
              xs_ref, wg_ref, wu_ref, wd_ref, out_ref, *, bt):
    h = pl.program_id(0)
    w = pl.program_id(1)

    @pl.when((h == 0) & (w == 0))
    def _init():
        out_ref[...] = jnp.zeros_like(out_ref)

    b = b_ref[w]
    lo = lo_ref[w]
    hi = hi_ref[w]

    @pl.when(hi > lo)
    def _compute():
        xb = xs_ref[pl.ds(b * bt, bt), :].astype(jnp.bfloat16)
        wg = wg_ref[0].astype(jnp.bfloat16)
        wu = wu_ref[0].astype(jnp.bfloat16)
        wd = wd_ref[0].astype(jnp.bfloat16)
        dn = (((1,), (1,)), ((), ()))
        gate = jax.lax.dot_general(xb, wg, dn,
                                   preferred_element_type=jnp.float32)
        up = jax.lax.dot_general(xb, wu, dn,
                                 preferred_element_type=jnp.float32)
        act = (gate * jax.nn.sigmoid(gate) * up).astype(jnp.bfloat16)
        contrib = jax.lax.dot_general(act, wd, dn,
                                      preferred_element_type=jnp.float32)
        iota = jax.lax.broadcasted_iota(jnp.int32, (bt, 1), 0)
        mask = ((iota >= lo) & (iota < hi)).astype(contrib.dtype)
        out_ref[pl.ds(b * bt, bt), :] += contrib * mask


def _grouped_swiglu(xs, Wgate, Wup, Wdown, b_ids, e_ids, lo, hi, *, bt, bh):
    S, D = xs.shape
    E, H, _ = Wgate.shape
    nh = H // bh
    nw = b_ids.shape[0]

    grid_spec = pltpu.PrefetchScalarGridSpec(
        num_scalar_prefetch=4,
        grid=(nh, nw),
        in_specs=[
            pl.BlockSpec((S, D), lambda h, w, b, e, l, u: (0, 0)),
            pl.BlockSpec((1, bh, D), lambda h, w, b, e, l, u: (e[w], h, 0)),
            pl.BlockSpec((1, bh, D), lambda h, w, b, e, l, u: (e[w], h, 0)),
            pl.BlockSpec((1, D, bh), lambda h, w, b, e, l, u: (e[w], 0, h)),
        ],
        out_specs=pl.BlockSpec((S, D), lambda h, w, b, e, l, u: (0, 0)),
    )
    return pl.pallas_call(
        functools.partial(_moe_body, bt=bt),
        grid_spec=grid_spec,
        out_shape=jax.ShapeDtypeStruct((S, D), jnp.float32),
        compiler_params=pltpu.CompilerParams(
            dimension_semantics=("arbitrary", "arbitrary"),
        ),
    )(b_ids, e_ids, lo, hi, xs, Wgate, Wup, Wdown)


def kernel(x, Wr, br, Wgate, Wup, Wdown):
    Bsz, S, D = x.shape
    E, H, _ = Wgate.shape
    BT = 128
    BH = 1024
    NB = S // BT
    NW = NB + E - 1

    xf = x.reshape(S, D)
    eid = _route(xf, Wr, br.reshape(1, E))[:, 0]

    counts = jnp.sum(eid[:, None] == jnp.arange(E, dtype=jnp.int32)[None, :],
                     axis=0).astype(jnp.int32)
    offs_end = jnp.cumsum(counts).astype(jnp.int32)
    order = jnp.argsort(eid, stable=True)

    blk = jnp.arange(NB, dtype=jnp.int32)
    e_first = jnp.searchsorted(offs_end, blk * BT, side='right').astype(jnp.int32)
    e_last = jnp.searchsorted(offs_end, blk * BT + (BT - 1),
                              side='right').astype(jnp.int32)
    k = e_last - e_first + 1
    item_end = jnp.cumsum(k).astype(jnp.int32)
    total = item_end[-1]

    w = jnp.arange(NW, dtype=jnp.int32)
    b_ids = jnp.minimum(jnp.searchsorted(item_end, w, side='right'),
                        NB - 1).astype(jnp.int32)
    item_start = item_end[b_ids] - k[b_ids]
    e_ids = jnp.minimum(e_first[b_ids] + (w - item_start),
                        E - 1).astype(jnp.int32)
    valid = w < total

    row_s = offs_end[e_ids] - counts[e_ids]
    row_e = offs_end[e_ids]
    lo = jnp.clip(row_s - b_ids * BT, 0, BT).astype(jnp.int32)
    hi = jnp.clip(row_e - b_ids * BT, 0, BT).astype(jnp.int32)
    hi = jnp.where(valid, hi, lo)

    xs = jnp.take(xf, order, axis=0, unique_indices=True)
    outs = _grouped_swiglu(xs, Wgate, Wup, Wdown, b_ids, e_ids, lo, hi,
                           bt=BT, bh=BH)
    out = jnp.zeros_like(outs).at[order].set(outs, unique_indices=True)
    return out.reshape(Bsz, S, D)


# speedup vs baseline: 3.7997x; 1.0718x over previous
import functools

import jax
import jax.numpy as jnp
from jax import lax
from jax.experimental import pallas as pl
from jax.experimental.pallas import tpu as pltpu
from jax.experimental.pallas import tpu_sc as plsc


def _routing_body(x_ref, wr_ref, br_ref, pos_ref, cnt_ref, *, sb):
    S = x_ref.shape[0]
    E = wr_ref.shape[0]
    logits = jax.lax.dot_general(
        wr_ref[...], x_ref[...],
        dimension_numbers=(((1,), (1,)), ((), ())),
        preferred_element_type=jnp.float32,
        precision=jax.lax.Precision.DEFAULT,
    )
    logits = logits + br_ref[...]
    row = jax.lax.broadcasted_iota(jnp.int32, (E, S), 0)
    maxv = jnp.max(logits, axis=0, keepdims=True)
    cand = jnp.where(logits >= maxv, row, E)
    eid = jnp.min(cand, axis=0, keepdims=True)
    onehot = (row == eid).astype(jnp.float32)

    cnt = jnp.sum(onehot, axis=1, keepdims=True)

    e0 = jax.lax.broadcasted_iota(jnp.int32, (E, E), 0)
    e1 = jax.lax.broadcasted_iota(jnp.int32, (E, E), 1)
    strict_lt = (e1 < e0).astype(jnp.float32)
    start = jax.lax.dot_general(strict_lt, cnt,
                                dimension_numbers=(((1,), (0,)), ((), ())),
                                preferred_element_type=jnp.float32,
                                precision=jax.lax.Precision.HIGHEST)

    t0 = jax.lax.broadcasted_iota(jnp.int32, (sb, sb), 0)
    t1 = jax.lax.broadcasted_iota(jnp.int32, (sb, sb), 1)
    triu = (t0 <= t1).astype(jnp.float32)
    nblk = S // sb

    base = jnp.zeros((E, 1), jnp.float32)
    for i in range(nblk):
        oh = onehot[:, i * sb:(i + 1) * sb]
        rank_incl = jax.lax.dot_general(
            oh, triu, dimension_numbers=(((1,), (0,)), ((), ())),
            preferred_element_type=jnp.float32)
        val = start + base + rank_incl - 1.0
        posb = jnp.sum(oh * val, axis=0, keepdims=True)
        pos_ref[:, i * sb:(i + 1) * sb] = posb.astype(jnp.int32)
        base = base + jnp.sum(oh, axis=1, keepdims=True)
    cnt_ref[...] = cnt.astype(jnp.int32)


def _route(xf, Wr, br2, *, sb=256):
    S = xf.shape[0]
    E = Wr.shape[0]
    return pl.pallas_call(
        functools.partial(_routing_body, sb=sb),
        out_shape=(jax.ShapeDtypeStruct((1, S), jnp.int32),
                   jax.ShapeDtypeStruct((E, 1), jnp.int32)),
    )(xf, Wr, br2)


def _sc_permute(src, pos, *, invert):
    S, D = src.shape
    num_cores, num_subcores = 2, 16
    nw = num_cores * num_subcores
    rpw = S // nw
    mesh = plsc.VectorSubcoreMesh(core_axis_name="c", subcore_axis_name="s")

    @functools.partial(
        pl.kernel, mesh=mesh,
        out_type=jax.ShapeDtypeStruct((S, D), jnp.float32),
        scratch_types=[
            pltpu.VMEM((rpw,), jnp.int32),
            pltpu.VMEM((rpw, D), jnp.float32),
            pltpu.SemaphoreType.DMA,
        ],
    )
    def k(src_hbm, pos_hbm, out_hbm, idx_v, rows_v, sem):
        wid = lax.axis_index("s") * num_cores + lax.axis_index("c")
        base = wid * rpw
        pltpu.sync_copy(pos_hbm.at[pl.ds(base, rpw)], idx_v)
        if invert:
            pltpu.async_copy(src_hbm.at[idx_v], rows_v, sem).wait()
            pltpu.sync_copy(rows_v, out_hbm.at[pl.ds(base, rpw)])
        else:
            pltpu.sync_copy(src_hbm.at[pl.ds(base, rpw)], rows_v)
            pltpu.async_copy(rows_v, out_hbm.at[idx_v], sem).wait()

    return k(src, pos)


def _moe_body(b_ref, e_ref, lo_ref, hi_ref,
              xs_ref, wg_ref, wu_ref, wd_ref, out_ref, *, bt):
    h = pl.program_id(0)
    w = pl.program_id(1)

    @pl.when((h == 0) & (w == 0))
    def _init():
        out_ref[...] = jnp.zeros_like(out_ref)

    b = b_ref[w]
    lo = lo_ref[w]
    hi = hi_ref[w]

    @pl.when(hi > lo)
    def _compute():
        xb = xs_ref[pl.ds(b * bt, bt), :]
        wg = wg_ref[0]
        wu = wu_ref[0]
        wd = wd_ref[0]
        dn = (((1,), (1,)), ((), ()))
        gate = jax.lax.dot_general(xb, wg, dn,
                                   preferred_element_type=jnp.float32,
                                   precision=jax.lax.Precision.DEFAULT)
        up = jax.lax.dot_general(xb, wu, dn,
                                 preferred_element_type=jnp.float32,
                                 precision=jax.lax.Precision.DEFAULT)
        act = gate * jax.nn.sigmoid(gate) * up
        contrib = jax.lax.dot_general(act, wd, dn,
                                      preferred_element_type=jnp.float32,
                                      precision=jax.lax.Precision.DEFAULT)
        iota = jax.lax.broadcasted_iota(jnp.int32, (bt, 1), 0)
        mask = ((iota >= lo) & (iota < hi)).astype(contrib.dtype)
        out_ref[pl.ds(b * bt, bt), :] += contrib * mask


def _grouped_swiglu(xs, Wgate, Wup, Wdown, b_ids, e_ids, lo, hi, *, bt, bh):
    S, D = xs.shape
    E, H, _ = Wgate.shape
    nh = H // bh
    nw = b_ids.shape[0]

    grid_spec = pltpu.PrefetchScalarGridSpec(
        num_scalar_prefetch=4,
        grid=(nh, nw),
        in_specs=[
            pl.BlockSpec((S, D), lambda h, w, b, e, l, u: (0, 0)),
            pl.BlockSpec((1, bh, D), lambda h, w, b, e, l, u: (e[w], h, 0)),
            pl.BlockSpec((1, bh, D), lambda h, w, b, e, l, u: (e[w], h, 0)),
            pl.BlockSpec((1, D, bh), lambda h, w, b, e, l, u: (e[w], 0, h)),
        ],
        out_specs=pl.BlockSpec((S, D), lambda h, w, b, e, l, u: (0, 0)),
    )
    return pl.pallas_call(
        functools.partial(_moe_body, bt=bt),
        grid_spec=grid_spec,
        out_shape=jax.ShapeDtypeStruct((S, D), jnp.float32),
        compiler_params=pltpu.CompilerParams(
            dimension_semantics=("arbitrary", "arbitrary"),
        ),
    )(b_ids, e_ids, lo, hi, xs, Wgate, Wup, Wdown)


def kernel(x, Wr, br, Wgate, Wup, Wdown):
    Bsz, S, D = x.shape
    E, H, _ = Wgate.shape
    BT = 128
    BH = 1024
    NB = S // BT
    NW = NB + E - 1

    xf = x.reshape(S, D)
    pos2, cnt2 = _route(xf, Wr, br.reshape(E, 1))
    pos = pos2.reshape(S)
    counts = cnt2.reshape(E)

    offs_end = jnp.cumsum(counts).astype(jnp.int32)

    blk = jnp.arange(NB, dtype=jnp.int32)
    e_first = jnp.searchsorted(offs_end, blk * BT, side='right').astype(jnp.int32)
    e_last = jnp.searchsorted(offs_end, blk * BT + (BT - 1),
                              side='right').astype(jnp.int32)
    k = e_last - e_first + 1
    item_end = jnp.cumsum(k).astype(jnp.int32)
    total = item_end[-1]

    w = jnp.arange(NW, dtype=jnp.int32)
    b_ids = jnp.minimum(jnp.searchsorted(item_end, w, side='right'),
                        NB - 1).astype(jnp.int32)
    item_start = item_end[b_ids] - k[b_ids]
    e_ids = jnp.minimum(e_first[b_ids] + (w - item_start),
                        E - 1).astype(jnp.int32)
    valid = w < total

    row_s = offs_end[e_ids] - counts[e_ids]
    row_e = offs_end[e_ids]
    lo = jnp.clip(row_s - b_ids * BT, 0, BT).astype(jnp.int32)
    hi = jnp.clip(row_e - b_ids * BT, 0, BT).astype(jnp.int32)
    hi = jnp.where(valid, hi, lo)

    xs = _sc_permute(xf, pos, invert=False)
    outs = _grouped_swiglu(xs, Wgate, Wup, Wdown, b_ids, e_ids, lo, hi,
                           bt=BT, bh=BH)
    out = _sc_permute(outs, pos, invert=True)
    return out.reshape(Bsz, S, D)


# speedup vs baseline: 5.2740x; 1.3880x over previous
import functools

import jax
import jax.numpy as jnp
from jax import lax
from jax.experimental import pallas as pl
from jax.experimental.pallas import tpu as pltpu
from jax.experimental.pallas import tpu_sc as plsc


def _routing_body(x_ref, wr_ref, br_ref, pos_ref, cnt_ref, *, sb):
    S = x_ref.shape[0]
    E = wr_ref.shape[0]
    logits = jax.lax.dot_general(
        wr_ref[...], x_ref[...],
        dimension_numbers=(((1,), (1,)), ((), ())),
        preferred_element_type=jnp.float32,
        precision=jax.lax.Precision.DEFAULT,
    )
    logits = logits + br_ref[...]
    row = jax.lax.broadcasted_iota(jnp.int32, (E, S), 0)
    maxv = jnp.max(logits, axis=0, keepdims=True)
    cand = jnp.where(logits >= maxv, row, E)
    eid = jnp.min(cand, axis=0, keepdims=True)
    onehot = (row == eid).astype(jnp.float32)

    cnt = jnp.sum(onehot, axis=1, keepdims=True)

    e0 = jax.lax.broadcasted_iota(jnp.int32, (E, E), 0)
    e1 = jax.lax.broadcasted_iota(jnp.int32, (E, E), 1)
    strict_lt = (e1 < e0).astype(jnp.float32)
    start = jax.lax.dot_general(strict_lt, cnt,
                                dimension_numbers=(((1,), (0,)), ((), ())),
                                preferred_element_type=jnp.float32,
                                precision=jax.lax.Precision.HIGHEST)

    t0 = jax.lax.broadcasted_iota(jnp.int32, (sb, sb), 0)
    t1 = jax.lax.broadcasted_iota(jnp.int32, (sb, sb), 1)
    triu = (t0 <= t1).astype(jnp.float32)
    nblk = S // sb

    base = jnp.zeros((E, 1), jnp.float32)
    for i in range(nblk):
        oh = onehot[:, i * sb:(i + 1) * sb]
        rank_incl = jax.lax.dot_general(
            oh, triu, dimension_numbers=(((1,), (0,)), ((), ())),
            preferred_element_type=jnp.float32)
        val = start + base + rank_incl - 1.0
        posb = jnp.sum(oh * val, axis=0, keepdims=True)
        pos_ref[:, i * sb:(i + 1) * sb] = posb.astype(jnp.int32)
        base = base + jnp.sum(oh, axis=1, keepdims=True)
    cnt_ref[...] = cnt.astype(jnp.int32)


def _route(xf, Wr, br2, *, sb=256):
    S = xf.shape[0]
    E = Wr.shape[0]
    return pl.pallas_call(
        functools.partial(_routing_body, sb=sb),
        out_shape=(jax.ShapeDtypeStruct((1, S), jnp.int32),
                   jax.ShapeDtypeStruct((E, 1), jnp.int32)),
    )(xf, Wr, br2)


def _sc_permute(src, pos, *, invert):
    S, D = src.shape
    num_cores, num_subcores = 2, 16
    nw = num_cores * num_subcores
    rpw = S // nw
    mesh = plsc.VectorSubcoreMesh(core_axis_name="c", subcore_axis_name="s")

    @functools.partial(
        pl.kernel, mesh=mesh,
        out_type=jax.ShapeDtypeStruct((S, D), jnp.float32),
        scratch_types=[
            pltpu.VMEM((rpw,), jnp.int32),
            pltpu.VMEM((rpw, D), jnp.float32),
            pltpu.SemaphoreType.DMA,
        ],
    )
    def k(src_hbm, pos_hbm, out_hbm, idx_v, rows_v, sem):
        wid = lax.axis_index("s") * num_cores + lax.axis_index("c")
        base = wid * rpw
        pltpu.sync_copy(pos_hbm.at[pl.ds(base, rpw)], idx_v)
        if invert:
            pltpu.async_copy(src_hbm.at[idx_v], rows_v, sem).wait()
            pltpu.sync_copy(rows_v, out_hbm.at[pl.ds(base, rpw)])
        else:
            pltpu.sync_copy(src_hbm.at[pl.ds(base, rpw)], rows_v)
            pltpu.async_copy(rows_v, out_hbm.at[idx_v], sem).wait()

    return k(src, pos)


def _moe_body(b_ref, e_ref, lo_ref, hi_ref,
              xs_ref, wg_ref, wu_ref, wd_ref, out_ref, *, bt):
    h = pl.program_id(0)
    w = pl.program_id(1)

    @pl.when((h == 0) & (w == 0))
    def _init():
        out_ref[...] = jnp.zeros_like(out_ref)

    b = b_ref[w]
    lo = lo_ref[w]
    hi = hi_ref[w]

    @pl.when(hi > lo)
    def _compute():
        xb = xs_ref[pl.ds(b * bt, bt), :]
        wg = wg_ref[0]
        wu = wu_ref[0]
        wd = wd_ref[0]
        dn = (((1,), (1,)), ((), ()))
        gate = jax.lax.dot_general(xb, wg, dn,
                                   preferred_element_type=jnp.float32,
                                   precision=jax.lax.Precision.DEFAULT)
        up = jax.lax.dot_general(xb, wu, dn,
                                 preferred_element_type=jnp.float32,
                                 precision=jax.lax.Precision.DEFAULT)
        act = gate * jax.nn.sigmoid(gate) * up
        contrib = jax.lax.dot_general(act, wd, dn,
                                      preferred_element_type=jnp.float32,
                                      precision=jax.lax.Precision.DEFAULT)
        iota = jax.lax.broadcasted_iota(jnp.int32, (bt, 1), 0)
        mask = ((iota >= lo) & (iota < hi)).astype(contrib.dtype)
        out_ref[pl.ds(b * bt, bt), :] += contrib * mask


def _grouped_swiglu(xs, Wgate, Wup, Wdown, b_ids, e_ids, lo, hi, *, bt, bh):
    S, D = xs.shape
    E, H, _ = Wgate.shape
    nh = H // bh
    nw = b_ids.shape[0]

    grid_spec = pltpu.PrefetchScalarGridSpec(
        num_scalar_prefetch=4,
        grid=(nh, nw),
        in_specs=[
            pl.BlockSpec((S, D), lambda h, w, b, e, l, u: (0, 0)),
            pl.BlockSpec((1, bh, D), lambda h, w, b, e, l, u: (e[w], h, 0)),
            pl.BlockSpec((1, bh, D), lambda h, w, b, e, l, u: (e[w], h, 0)),
            pl.BlockSpec((1, D, bh), lambda h, w, b, e, l, u: (e[w], 0, h)),
        ],
        out_specs=pl.BlockSpec((S, D), lambda h, w, b, e, l, u: (0, 0)),
    )
    return pl.pallas_call(
        functools.partial(_moe_body, bt=bt),
        grid_spec=grid_spec,
        out_shape=jax.ShapeDtypeStruct((S, D), jnp.float32),
        compiler_params=pltpu.CompilerParams(
            dimension_semantics=("arbitrary", "arbitrary"),
        ),
    )(b_ids, e_ids, lo, hi, xs, Wgate, Wup, Wdown)


def kernel(x, Wr, br, Wgate, Wup, Wdown):
    Bsz, S, D = x.shape
    E, H, _ = Wgate.shape
    BT = 256
    BH = 1024
    NB = S // BT
    NW = NB + E - 1

    xf = x.reshape(S, D)
    pos2, cnt2 = _route(xf, Wr, br.reshape(E, 1))
    pos = pos2.reshape(S)
    counts = cnt2.reshape(E)

    offs_end = jnp.cumsum(counts).astype(jnp.int32)

    blk = jnp.arange(NB, dtype=jnp.int32)
    e_first = jnp.searchsorted(offs_end, blk * BT, side='right').astype(jnp.int32)
    e_last = jnp.searchsorted(offs_end, blk * BT + (BT - 1),
                              side='right').astype(jnp.int32)
    k = e_last - e_first + 1
    item_end = jnp.cumsum(k).astype(jnp.int32)
    total = item_end[-1]

    w = jnp.arange(NW, dtype=jnp.int32)
    b_ids = jnp.minimum(jnp.searchsorted(item_end, w, side='right'),
                        NB - 1).astype(jnp.int32)
    item_start = item_end[b_ids] - k[b_ids]
    e_ids = jnp.minimum(e_first[b_ids] + (w - item_start),
                        E - 1).astype(jnp.int32)
    valid = w < total

    row_s = offs_end[e_ids] - counts[e_ids]
    row_e = offs_end[e_ids]
    lo = jnp.clip(row_s - b_ids * BT, 0, BT).astype(jnp.int32)
    hi = jnp.clip(row_e - b_ids * BT, 0, BT).astype(jnp.int32)
    hi = jnp.where(valid, hi, lo)

    xs = _sc_permute(xf, pos, invert=False)
    outs = _grouped_swiglu(xs, Wgate, Wup, Wdown, b_ids, e_ids, lo, hi,
                           bt=BT, bh=BH)
    out = _sc_permute(outs, pos, invert=True)
    return out.reshape(Bsz, S, D)
